# Initial kernel scaffold; baseline (speedup 1.0000x reference)
import jax
import jax.numpy as jnp
from jax.experimental import pallas as pl


def kernel(h, edge_index, edge_w, W1, b1, g1, be1, W2, b2, g2, be2, eps_list):
    raise NotImplementedError("write your pallas kernel here")


# speedup vs baseline: 3.7304x; 3.7304x over previous
import functools

import jax
import jax.numpy as jnp
from jax import lax
from jax.experimental import pallas as pl
from jax.experimental.pallas import tpu as pltpu
from jax.experimental.pallas import tpu_sc as plsc

_NC = 2
_NS = 16
_LANES = 16
_CHUNK = 80


@functools.partial(jax.jit, static_argnames=("n", "e", "d"))
def _segment_sum_sc(h, src, dst, ew, zeros, *, n, e, d):
    nw = _NC * _NS
    edges_per_w = e // nw
    chunks = edges_per_w // _CHUNK
    rows_main = (n // _NS) // 8 * 8
    rows_last = n - (_NS - 1) * rows_main
    mesh = plsc.VectorSubcoreMesh(core_axis_name="c", subcore_axis_name="s")

    @functools.partial(
        pl.kernel,
        mesh=mesh,
        out_type=jax.ShapeDtypeStruct((_NC, n, d), jnp.float32),
        scratch_types=[
            pltpu.VMEM((_CHUNK,), jnp.int32),
            pltpu.VMEM((_CHUNK,), jnp.int32),
            pltpu.VMEM((_CHUNK,), jnp.float32),
            pltpu.VMEM((_CHUNK, d), jnp.float32),
            pltpu.VMEM_SHARED((n, d), jnp.float32),
            pltpu.SemaphoreType.DMA,
        ],
    )
    def k(h_hbm, src_hbm, dst_hbm, ew_hbm, z_hbm, out_hbm,
          sidx_v, didx_v, ew_v, rows_v, acc_sh, sem):
        cid = lax.axis_index("c")
        sid = lax.axis_index("s")

        rbase = sid * rows_main

        @pl.when(sid < _NS - 1)
        def _():
            pltpu.sync_copy(z_hbm.at[pl.ds(rbase, rows_main)],
                            acc_sh.at[pl.ds(rbase, rows_main)])

        @pl.when(sid == _NS - 1)
        def _():
            pltpu.sync_copy(z_hbm.at[pl.ds(rbase, rows_last)],
                            acc_sh.at[pl.ds(rbase, rows_last)])

        plsc.subcore_barrier()

        ebase = (cid * _NS + sid) * edges_per_w

        def chunk_body(i, carry):
            off = ebase + i * _CHUNK
            pltpu.sync_copy(src_hbm.at[pl.ds(off, _CHUNK)], sidx_v)
            pltpu.sync_copy(dst_hbm.at[pl.ds(off, _CHUNK)], didx_v)
            pltpu.sync_copy(ew_hbm.at[pl.ds(off, _CHUNK)], ew_v)
            pltpu.async_copy(h_hbm.at[sidx_v], rows_v, sem).wait()

            def scale_body(j, c2):
                wv = ew_v[pl.ds(j * _LANES, _LANES)]
                for k in range(_LANES):
                    w = wv[k]
                    row = j * _LANES + k
                    for g in range(d // _LANES):
                        blk = rows_v[row, pl.ds(g * _LANES, _LANES)]
                        rows_v[row, pl.ds(g * _LANES, _LANES)] = w * blk
                return c2

            lax.fori_loop(0, _CHUNK // _LANES, scale_body, 0)
            pltpu.sync_copy(rows_v, acc_sh.at[didx_v], add=True)
            return carry

        lax.fori_loop(0, chunks, chunk_body, 0)
        plsc.subcore_barrier()

        @pl.when(sid < _NS - 1)
        def _():
            pltpu.sync_copy(acc_sh.at[pl.ds(rbase, rows_main)],
                            out_hbm.at[cid, pl.ds(rbase, rows_main)])

        @pl.when(sid == _NS - 1)
        def _():
            pltpu.sync_copy(acc_sh.at[pl.ds(rbase, rows_last)],
                            out_hbm.at[cid, pl.ds(rbase, rows_last)])

    return k(h, src, dst, ew, zeros)


def _mlp_body(eps_ref, h_ref, parts_ref, w1_ref, b1_ref, g1_ref, be1_ref,
              w2_ref, b2_ref, g2_ref, be2_ref, out_ref):
    x = (1.0 + eps_ref[0, 0]) * h_ref[...] + parts_ref[0] + parts_ref[1]
    t = jnp.dot(x, w1_ref[...], preferred_element_type=jnp.float32) + b1_ref[...]
    mu = jnp.mean(t, axis=0, keepdims=True)
    var = jnp.mean((t - mu) ** 2, axis=0, keepdims=True)
    t = g1_ref[...] * (t - mu) * lax.rsqrt(var + 1e-5) + be1_ref[...]
    t = jnp.maximum(t, 0.0)
    t = jnp.maximum(
        jnp.dot(t, w2_ref[...], preferred_element_type=jnp.float32) + b2_ref[...],
        0.0)
    mu2 = jnp.mean(t, axis=0, keepdims=True)
    var2 = jnp.mean((t - mu2) ** 2, axis=0, keepdims=True)
    t = g2_ref[...] * (t - mu2) * lax.rsqrt(var2 + 1e-5) + be2_ref[...]
    out_ref[...] = jnp.maximum(t, 0.0)


def _mlp_tc(eps, h, parts, w1, b1, g1, be1, w2, b2, g2, be2):
    n, d = h.shape
    return pl.pallas_call(
        _mlp_body,
        out_shape=jax.ShapeDtypeStruct((n, d), jnp.float32),
    )(eps, h, parts, w1, b1, g1, be1, w2, b2, g2, be2)


def kernel(h, edge_index, edge_w, W1, b1, g1, be1, W2, b2, g2, be2, eps_list):
    n, d = h.shape
    e = edge_w.shape[0]
    nlayers = W1.shape[0]
    src = edge_index[0]
    dst = edge_index[1]
    zeros = jnp.zeros((n, d), jnp.float32)
    for l in range(nlayers):
        parts = _segment_sum_sc(h, src, dst, edge_w, zeros, n=n, e=e, d=d)
        h = _mlp_tc(jnp.reshape(eps_list[l], (1, 1)), h, parts,
                    W1[l], jnp.reshape(b1[l], (1, d)), jnp.reshape(g1[l], (1, d)),
                    jnp.reshape(be1[l], (1, d)), W2[l], jnp.reshape(b2[l], (1, d)),
                    jnp.reshape(g2[l], (1, d)), jnp.reshape(be2[l], (1, d)))
    return h
